# baseline (speedup 1.0000x reference)
import functools

import jax
import jax.numpy as jnp
from jax import lax
from jax.experimental import pallas as pl
from jax.experimental.pallas import tpu as pltpu
from jax.experimental.pallas import tpu_sc as plsc

_N0, _N1, _E, _H, _NI = 10000, 20000, 160000, 256, 16
_NC, _NS = 2, 16
_NW = _NC * _NS
_EB = 128
_NBLK = _E // _EB
_NBF = _NBLK // _NW
_NBR = _NBLK - _NBF * _NW
_SBF = _NBLK // _NS
_SBR = _NBLK - _SBF * _NS

_f32 = jnp.float32


def _silu(x):
    return x * jax.nn.sigmoid(x)


def _mesh():
    return plsc.VectorSubcoreMesh(
        core_axis_name="c", subcore_axis_name="s",
        num_cores=_NC, num_subcores=_NS)


_HW = _H // 2


def _proj(x, ws, bs, br=1024):
    n = x.shape[0]
    nw = len(ws)

    def body(x_ref, *refs):
        w_refs = refs[:nw]
        b_refs = refs[nw:2 * nw]
        o_refs = refs[2 * nw:]
        xb = x_ref[...]
        for j in range(nw):
            r = (jnp.dot(xb, w_refs[j][...], preferred_element_type=_f32)
                 + b_refs[j][...])
            lo = jax.lax.bitcast_convert_type(r[:, :_HW], jnp.int32)
            hi = jax.lax.bitcast_convert_type(r[:, _HW:], jnp.int32)
            lo16 = jnp.bitwise_and((lo + 0x8000) >> 16, 0xFFFF)
            hi16 = jnp.bitwise_and(hi + 0x8000, jnp.int32(-65536))
            o_refs[j][...] = jnp.bitwise_or(lo16, hi16)

    in_specs = [pl.BlockSpec((br, _H), lambda i: (i, 0))]
    in_specs += [pl.BlockSpec((_H, _H), lambda i: (0, 0))] * nw
    in_specs += [pl.BlockSpec((1, _H), lambda i: (0, 0))] * nw
    out_specs = [pl.BlockSpec((br, _HW), lambda i: (i, 0))] * nw
    return pl.pallas_call(
        body, grid=(pl.cdiv(n, br),),
        in_specs=in_specs, out_specs=out_specs,
        out_shape=[jax.ShapeDtypeStruct((n, _HW), jnp.int32)] * nw,
    )(x, *ws, *bs)


def _edge_mlp(ga, gb, inv, w1i, w2, b2, wip, bip, be=1280):
    def body(ga_ref, gb_ref, inv_ref, w1i_ref, w2_ref, b2_ref, wip_ref,
             bip_ref, y_ref):
        ai = ga_ref[...]
        bi_ = gb_ref[...]
        himask = jnp.int32(-65536)
        lo = (jax.lax.bitcast_convert_type(ai << 16, _f32)
              + jax.lax.bitcast_convert_type(bi_ << 16, _f32))
        hi = (jax.lax.bitcast_convert_type(jnp.bitwise_and(ai, himask), _f32)
              + jax.lax.bitcast_convert_type(jnp.bitwise_and(bi_, himask),
                                             _f32))
        t = (jnp.concatenate([lo, hi], axis=1)
             + jnp.dot(inv_ref[...], w1i_ref[...], preferred_element_type=_f32))
        m = _silu(t)
        m2 = _silu(jnp.dot(m.astype(jnp.bfloat16), w2_ref[...],
                           preferred_element_type=_f32) + b2_ref[...])
        g = jax.nn.sigmoid(
            jnp.dot(m2.astype(jnp.bfloat16), wip_ref[...],
                    preferred_element_type=_f32) + bip_ref[...])
        y_ref[...] = m2 * g[:, 0:1]

    in_specs = [
        pl.BlockSpec((be, _HW), lambda i: (i, 0)),
        pl.BlockSpec((be, _HW), lambda i: (i, 0)),
        pl.BlockSpec((be, _NI), lambda i: (i, 0)),
        pl.BlockSpec((_NI, _H), lambda i: (0, 0)),
        pl.BlockSpec((_H, _H), lambda i: (0, 0)),
        pl.BlockSpec((1, _H), lambda i: (0, 0)),
        pl.BlockSpec((_H, 128), lambda i: (0, 0)),
        pl.BlockSpec((1, 128), lambda i: (0, 0)),
    ]
    return pl.pallas_call(
        body, grid=(_E // be,),
        in_specs=in_specs,
        out_specs=pl.BlockSpec((be, _H), lambda i: (i, 0)),
        out_shape=jax.ShapeDtypeStruct((_E, _H), _f32),
    )(ga, gb, inv, w1i, w2, b2, wip, bip)


_EPW = _E // _NW
_SB = 3 * _EB
_NSB = _EPW // _SB
_REM = _EPW - _NSB * _SB


def _sc_gather_all(tabs_and_idx):
    n_t = len(tabs_and_idx)

    @functools.partial(
        pl.kernel,
        out_type=[jax.ShapeDtypeStruct((_E, _HW), jnp.int32)] * (2 * n_t),
        mesh=_mesh(),
        scratch_types=[
            pltpu.VMEM((_SB,), jnp.int32),
            pltpu.VMEM((_SB,), jnp.int32),
            pltpu.VMEM((_SB, _HW), jnp.int32),
            pltpu.VMEM((_SB, _HW), jnp.int32),
            pltpu.SemaphoreType.DMA,
            pltpu.SemaphoreType.DMA,
            pltpu.SemaphoreType.DMA,
            pltpu.SemaphoreType.DMA,
            pltpu.SemaphoreType.DMA,
        ],
    )
    def k(*refs):
        ins = refs[:4 * n_t]
        outs = refs[4 * n_t:4 * n_t + 2 * n_t]
        (idx0v, idx1v, buf0, buf1, si0, si1, sg, sw0,
         sw1) = refs[4 * n_t + 2 * n_t:]
        isems = (si0, si1)
        w = lax.axis_index("s") * _NC + lax.axis_index("c")
        base = w * _EPW
        bufs = (buf0, buf1)
        idxvs = (idx0v, idx1v)
        wsems = (sw0, sw1)

        jobs = []
        for ty in range(n_t):
            a_hbm, i0_hbm, b_hbm, i1_hbm = ins[4 * ty:4 * ty + 4]
            jobs.append((a_hbm, i0_hbm, outs[2 * ty]))
            jobs.append((b_hbm, i1_hbm, outs[2 * ty + 1]))

        for tab, i_hbm, out_hbm in jobs:
            pend = [None, None]
            ipend = [None, None]
            e00 = pl.multiple_of(base, 8)
            ipend[0] = pltpu.async_copy(i_hbm.at[pl.ds(e00, _SB)], idx0v,
                                        si0)
            for sb in range(_NSB):
                p = sb % 2
                buf = bufs[p]
                if pend[p] is not None:
                    pend[p].wait()
                ipend[p].wait()
                gs = [pltpu.async_copy(
                    tab.at[idxvs[p].at[pl.ds(t * _EB, _EB)]],
                    buf.at[pl.ds(t * _EB, _EB)], sg)
                    for t in range(_SB // _EB)]
                if sb + 1 < _NSB:
                    e1 = pl.multiple_of(base + (sb + 1) * _SB, 8)
                    ipend[1 - p] = pltpu.async_copy(
                        i_hbm.at[pl.ds(e1, _SB)], idxvs[1 - p],
                        isems[1 - p])
                for g in gs:
                    g.wait()
                e0 = pl.multiple_of(base + sb * _SB, 8)
                pend[p] = pltpu.async_copy(
                    buf, out_hbm.at[pl.ds(e0, _SB)], wsems[p])
            for p in range(2):
                if pend[p] is not None:
                    pend[p].wait()
            er = pl.multiple_of(base + _NSB * _SB, 8)
            pltpu.sync_copy(i_hbm.at[pl.ds(er, _REM)],
                            idx0v.at[pl.ds(0, _REM)])
            pltpu.async_copy(tab.at[idx0v.at[pl.ds(0, _REM)]],
                             buf0.at[pl.ds(0, _REM)], sg).wait()
            pltpu.sync_copy(buf0.at[pl.ds(0, _REM)],
                            out_hbm.at[pl.ds(er, _REM)])

    flat = []
    for t4 in tabs_and_idx:
        flat.extend(t4)
    return k(*flat)


_NBPS = _NBLK // _NS
_NBPX = _NBLK - _NBPS * _NS
_CW = _H // _NC
_N0P = 10112
_N0ZR = _N0P // _NS
_N0PAD = 10240
_N1HALF = 10112
_N1ACC = 10240
_N1ZR = _N1ACC // _NS
_N1PAD = 20480


def _scatter_blocks(y_hbm, i_hbm, col0, accum, ibufs, ybufs, lsems, ssems,
                    sub, i_is_2d):
    def load_idx(blk, ib, sem):
        if i_is_2d:
            return pltpu.async_copy(i_hbm.at[blk], ib.at[0], sem)
        return pltpu.async_copy(i_hbm.at[pl.ds(blk * _EB, _EB)], ib.at[0],
                                sem)

    pend = [None, None]
    for j in range(_NBPS):
        p = j % 2
        ib, yb = ibufs[p], ybufs[p]
        if pend[p] is not None:
            pend[p].wait()
        blk = sub * _NBPS + j
        e0 = pl.multiple_of(blk * _EB, _EB)
        la = pltpu.async_copy(y_hbm.at[pl.ds(e0, _EB), pl.ds(col0, _CW)],
                              yb, lsems[p])
        li = load_idx(blk, ib, lsems[p])
        la.wait()
        li.wait()
        pend[p] = pltpu.async_copy(yb, accum.at[ib.at[0]], ssems[p],
                                   add=True)
    for p in range(2):
        if pend[p] is not None:
            pend[p].wait()

    @pl.when(sub < _NBPX)
    def _():
        blk = _NS * _NBPS + sub
        e0 = pl.multiple_of(blk * _EB, _EB)
        pltpu.sync_copy(y_hbm.at[pl.ds(e0, _EB), pl.ds(col0, _CW)],
                        ybufs[0])
        load_idx(blk, ibufs[0], lsems[0]).wait()
        pltpu.sync_copy(ybufs[0], accum.at[ibufs[0].at[0]], add=True)


def _sc_scatter_all(y0, i0, y1, i1, y11, ilo, ihi, zeros):
    @functools.partial(
        pl.kernel,
        out_type=[jax.ShapeDtypeStruct((_N0PAD, _H), _f32),
                  jax.ShapeDtypeStruct((_N0PAD, _H), _f32),
                  jax.ShapeDtypeStruct((_N1PAD, _H), _f32)],
        mesh=_mesh(),
        scratch_types=[
            pltpu.VMEM((1, _EB), jnp.int32),
            pltpu.VMEM((1, _EB), jnp.int32),
            pltpu.VMEM((_EB, _CW), _f32),
            pltpu.VMEM((_EB, _CW), _f32),
            pltpu.VMEM_SHARED((_N1ACC, _CW), _f32),
            pltpu.SemaphoreType.DMA,
            pltpu.SemaphoreType.DMA,
            pltpu.SemaphoreType.DMA,
            pltpu.SemaphoreType.DMA,
        ],
    )
    def k(z_hbm, y0_hbm, i0_hbm, y1_hbm, i1_hbm, y11_hbm, ilo_hbm, ihi_hbm,
          m0_hbm, m1_hbm, m11_hbm,
          ib0, ib1, yb0, yb1, accum, sl0, sl1, ss0, ss1):
        core = lax.axis_index("c")
        sub = lax.axis_index("s")
        col0 = pl.multiple_of(core * _CW, _CW)

        sweeps = (
            (y0_hbm, i0_hbm, False, m0_hbm, 0),
            (y1_hbm, i1_hbm, False, m1_hbm, 0),
            (y11_hbm, ilo_hbm, True, m11_hbm, 0),
            (y11_hbm, ihi_hbm, True, m11_hbm, _N1HALF),
        )
        for y_hbm, i_hbm, is2d, m_hbm, mbase in sweeps:
            r0 = pl.multiple_of(sub * _N1ZR, 8)
            pltpu.sync_copy(z_hbm.at[pl.ds(0, _N1ZR)],
                            accum.at[pl.ds(r0, _N1ZR)])
            plsc.subcore_barrier()
            _scatter_blocks(y_hbm, i_hbm, col0, accum, (ib0, ib1),
                            (yb0, yb1), (sl0, sl1), (ss0, ss1), sub,
                            i_is_2d=is2d)
            plsc.subcore_barrier()

            @pl.when(sub < _NS - 1)
            def _():
                pltpu.sync_copy(
                    accum.at[pl.ds(r0, _N1ZR)],
                    m_hbm.at[pl.ds(mbase + r0, _N1ZR), pl.ds(col0, _CW)])

            @pl.when(sub == _NS - 1)
            def _():
                last = pl.multiple_of((_NS - 1) * _N1ZR, 8)
                rows = _N1HALF - (_NS - 1) * _N1ZR
                pltpu.sync_copy(
                    accum.at[pl.ds(last, rows)],
                    m_hbm.at[pl.ds(mbase + last, rows), pl.ds(col0, _CW)])

            plsc.subcore_barrier()

    return k(zeros, y0, i0, y1, i1, y11, ilo, ihi)


def _remap_n1(i2d):
    def body(i_ref, lo_ref, hi_ref):
        v = i_ref[...]
        trash = _N1HALF + jnp.bitwise_and(v, 127)
        lo_ref[...] = jnp.where(v < _N1HALF, v, trash)
        hi_ref[...] = jnp.where(v >= _N1HALF, v - _N1HALF, trash)

    return pl.pallas_call(
        body,
        out_shape=[jax.ShapeDtypeStruct((_NBLK, _EB), jnp.int32)] * 2,
    )(i2d)


def _update(parts, w_parts, b1, w2, b2, br=1024):
    n = parts[0].shape[0]
    k = len(parts)

    def body(*refs):
        p_refs = refs[:k]
        wp_refs = refs[k:2 * k]
        b1_ref, w2_ref, b2_ref, o_ref = refs[2 * k:]
        s = jnp.dot(p_refs[0][...], wp_refs[0][...],
                    preferred_element_type=_f32)
        for j in range(1, k):
            s = s + jnp.dot(p_refs[j][...], wp_refs[j][...],
                            preferred_element_type=_f32)
        h = _silu(s + b1_ref[...])
        o_ref[...] = (jnp.dot(h, w2_ref[...], preferred_element_type=_f32)
                      + b2_ref[...] + p_refs[0][...])

    in_specs = [pl.BlockSpec((br, _H), lambda i: (i, 0))] * k
    in_specs += [pl.BlockSpec((_H, _H), lambda i: (0, 0))] * k
    in_specs += [pl.BlockSpec((1, _H), lambda i: (0, 0)),
                 pl.BlockSpec((_H, _H), lambda i: (0, 0)),
                 pl.BlockSpec((1, _H), lambda i: (0, 0))]
    return pl.pallas_call(
        body, grid=(pl.cdiv(n, br),),
        in_specs=in_specs,
        out_specs=pl.BlockSpec((br, _H), lambda i: (i, 0)),
        out_shape=jax.ShapeDtypeStruct((n, _H), _f32),
    )(*parts, *w_parts, b1, w2, b2)


def kernel(x_0, x_1, adj_0_0, adj_1_0, adj_1_1, inv_0_0, inv_1_0, inv_1_1,
           msg_W1_0_0, msg_b1_0_0, msg_W2_0_0, msg_b2_0_0, inf_W_0_0, inf_b_0_0,
           msg_W1_1_0, msg_b1_1_0, msg_W2_1_0, msg_b2_1_0, inf_W_1_0, inf_b_1_0,
           msg_W1_1_1, msg_b1_1_1, msg_W2_1_1, msg_b2_1_1, inf_W_1_1, inf_b_1_1,
           upd_W1_0, upd_b1_0, upd_W2_0, upd_b2_0,
           upd_W1_1, upd_b1_1, upd_W2_1, upd_b2_1):
    i32 = jnp.int32
    idx = {
        "0_0": (adj_0_0[0].astype(i32), adj_0_0[1].astype(i32)),
        "1_0": (adj_1_0[0].astype(i32), adj_1_0[1].astype(i32)),
        "1_1": (adj_1_1[0].astype(i32), adj_1_1[1].astype(i32)),
    }
    w1 = {"0_0": msg_W1_0_0, "1_0": msg_W1_1_0, "1_1": msg_W1_1_1}
    b1 = {"0_0": msg_b1_0_0, "1_0": msg_b1_1_0, "1_1": msg_b1_1_1}
    w2 = {"0_0": msg_W2_0_0, "1_0": msg_W2_1_0, "1_1": msg_W2_1_1}
    b2 = {"0_0": msg_b2_0_0, "1_0": msg_b2_1_0, "1_1": msg_b2_1_1}
    wi = {"0_0": inf_W_0_0, "1_0": inf_W_1_0, "1_1": inf_W_1_1}
    bi = {"0_0": inf_b_0_0, "1_0": inf_b_1_0, "1_1": inf_b_1_1}
    inv = {"0_0": inv_0_0, "1_0": inv_1_0, "1_1": inv_1_1}

    w1s = {a: w1[a][:_H] for a in w1}
    w1r = {a: w1[a][_H:2 * _H] for a in w1}
    w1i = {a: w1[a][2 * _H:] for a in w1}
    b1row = {a: b1[a][None, :] for a in b1}
    b2row = {a: b2[a][None, :] for a in b2}
    w2b = {a: w2[a].astype(jnp.bfloat16) for a in w2}
    wip = {a: jnp.pad(wi[a], ((0, 0), (0, 127))).astype(jnp.bfloat16)
           for a in wi}
    bip = {a: jnp.pad(bi[a], (0, 127))[None, :] for a in bi}
    zrow = jnp.zeros((1, _H), _f32)

    A00, B00, B10 = _proj(
        x_0, [w1s["0_0"], w1r["0_0"], w1r["1_0"]],
        [zrow, b1row["0_0"], b1row["1_0"]])
    A10, A11, B11 = _proj(
        x_1, [w1s["1_0"], w1s["1_1"], w1r["1_1"]],
        [zrow, zrow, b1row["1_1"]])
    tabs = {"0_0": (A00, B00), "1_0": (A10, B10), "1_1": (A11, B11)}

    g = _sc_gather_all([
        (tabs[a][0], idx[a][0], tabs[a][1], idx[a][1])
        for a in ("0_0", "1_0", "1_1")])
    y = {}
    for j, a in enumerate(("0_0", "1_0", "1_1")):
        y[a] = _edge_mlp(g[2 * j], g[2 * j + 1], inv[a], w1i[a], w2b[a],
                         b2row[a], wip[a], bip[a])

    zeros = jnp.zeros((_N1ZR, _CW), _f32)
    ilo, ihi = _remap_n1(idx["1_1"][1].reshape(_NBLK, _EB))
    mes00, mes10, mes11 = _sc_scatter_all(
        y["0_0"], idx["0_0"][1], y["1_0"], idx["1_0"][1],
        y["1_1"], ilo, ihi, zeros)

    u1_0 = [upd_W1_0[:_H], upd_W1_0[_H:2 * _H], upd_W1_0[2 * _H:]]
    out0 = _update([x_0, mes00, mes10], u1_0, upd_b1_0[None, :],
                   upd_W2_0, upd_b2_0[None, :])
    u1_1 = [upd_W1_1[:_H], upd_W1_1[_H:]]
    out1 = _update([x_1, mes11], u1_1, upd_b1_1[None, :],
                   upd_W2_1, upd_b2_1[None, :])
    return (out0, out1)


# speedup vs baseline: 1.3278x; 1.3278x over previous
import functools

import jax
import jax.numpy as jnp
from jax import lax
from jax.experimental import pallas as pl
from jax.experimental.pallas import tpu as pltpu
from jax.experimental.pallas import tpu_sc as plsc

_N0, _N1, _E, _H, _NI = 10000, 20000, 160000, 256, 16
_NC, _NS = 2, 16
_NW = _NC * _NS
_EB = 128
_NBLK = _E // _EB
_NBF = _NBLK // _NW
_NBR = _NBLK - _NBF * _NW
_SBF = _NBLK // _NS
_SBR = _NBLK - _SBF * _NS

_f32 = jnp.float32


def _silu(x):
    return x * jax.nn.sigmoid(x)


def _mesh():
    return plsc.VectorSubcoreMesh(
        core_axis_name="c", subcore_axis_name="s",
        num_cores=_NC, num_subcores=_NS)


_HW = _H // 2


def _proj(x, ws, bs, br=1024):
    n = x.shape[0]
    nw = len(ws)

    def body(x_ref, *refs):
        w_refs = refs[:nw]
        b_refs = refs[nw:2 * nw]
        o_refs = refs[2 * nw:]
        xb = x_ref[...]
        for j in range(nw):
            r = (jnp.dot(xb, w_refs[j][...], preferred_element_type=_f32)
                 + b_refs[j][...])
            lo = jax.lax.bitcast_convert_type(r[:, :_HW], jnp.int32)
            hi = jax.lax.bitcast_convert_type(r[:, _HW:], jnp.int32)
            lo16 = jnp.bitwise_and((lo + 0x8000) >> 16, 0xFFFF)
            hi16 = jnp.bitwise_and(hi + 0x8000, jnp.int32(-65536))
            o_refs[j][...] = jnp.bitwise_or(lo16, hi16)

    in_specs = [pl.BlockSpec((br, _H), lambda i: (i, 0))]
    in_specs += [pl.BlockSpec((_H, _H), lambda i: (0, 0))] * nw
    in_specs += [pl.BlockSpec((1, _H), lambda i: (0, 0))] * nw
    out_specs = [pl.BlockSpec((br, _HW), lambda i: (i, 0))] * nw
    return pl.pallas_call(
        body, grid=(pl.cdiv(n, br),),
        in_specs=in_specs, out_specs=out_specs,
        out_shape=[jax.ShapeDtypeStruct((n, _HW), jnp.int32)] * nw,
    )(x, *ws, *bs)


def _edge_mlp(ga, gb, inv, w1i, w2, b2, wip, bip, be=1280):
    def body(ga_ref, gb_ref, inv_ref, w1i_ref, w2_ref, b2_ref, wip_ref,
             bip_ref, y_ref):
        ai = ga_ref[...]
        bi_ = gb_ref[...]
        himask = jnp.int32(-65536)
        lo = (jax.lax.bitcast_convert_type(ai << 16, _f32)
              + jax.lax.bitcast_convert_type(bi_ << 16, _f32))
        hi = (jax.lax.bitcast_convert_type(jnp.bitwise_and(ai, himask), _f32)
              + jax.lax.bitcast_convert_type(jnp.bitwise_and(bi_, himask),
                                             _f32))
        t = (jnp.concatenate([lo, hi], axis=1)
             + jnp.dot(inv_ref[...], w1i_ref[...], preferred_element_type=_f32))
        m = _silu(t)
        m2 = _silu(jnp.dot(m.astype(jnp.bfloat16), w2_ref[...],
                           preferred_element_type=_f32) + b2_ref[...])
        g = jax.nn.sigmoid(
            jnp.dot(m2.astype(jnp.bfloat16), wip_ref[...],
                    preferred_element_type=_f32) + bip_ref[...])
        y_ref[...] = m2 * g[:, 0:1]

    in_specs = [
        pl.BlockSpec((be, _HW), lambda i: (i, 0)),
        pl.BlockSpec((be, _HW), lambda i: (i, 0)),
        pl.BlockSpec((be, _NI), lambda i: (i, 0)),
        pl.BlockSpec((_NI, _H), lambda i: (0, 0)),
        pl.BlockSpec((_H, _H), lambda i: (0, 0)),
        pl.BlockSpec((1, _H), lambda i: (0, 0)),
        pl.BlockSpec((_H, 128), lambda i: (0, 0)),
        pl.BlockSpec((1, 128), lambda i: (0, 0)),
    ]
    return pl.pallas_call(
        body, grid=(_E // be,),
        in_specs=in_specs,
        out_specs=pl.BlockSpec((be, _H), lambda i: (i, 0)),
        out_shape=jax.ShapeDtypeStruct((_E, _H), _f32),
    )(ga, gb, inv, w1i, w2, b2, wip, bip)


_EPW = _E // _NW
_SB = 3 * _EB
_NSB = _EPW // _SB
_REM = _EPW - _NSB * _SB


def _sc_gather_all(tabs_and_idx):
    n_t = len(tabs_and_idx)

    @functools.partial(
        pl.kernel,
        out_type=[jax.ShapeDtypeStruct((_E, _HW), jnp.int32)] * (2 * n_t),
        mesh=_mesh(),
        scratch_types=[
            pltpu.VMEM((_SB,), jnp.int32),
            pltpu.VMEM((_SB,), jnp.int32),
            pltpu.VMEM((_SB, _HW), jnp.int32),
            pltpu.VMEM((_SB, _HW), jnp.int32),
            pltpu.SemaphoreType.DMA,
            pltpu.SemaphoreType.DMA,
            pltpu.SemaphoreType.DMA,
            pltpu.SemaphoreType.DMA,
            pltpu.SemaphoreType.DMA,
        ],
    )
    def k(*refs):
        ins = refs[:4 * n_t]
        outs = refs[4 * n_t:4 * n_t + 2 * n_t]
        (idx0v, idx1v, buf0, buf1, si0, si1, sg, sw0,
         sw1) = refs[4 * n_t + 2 * n_t:]
        isems = (si0, si1)
        w = lax.axis_index("s") * _NC + lax.axis_index("c")
        base = w * _EPW
        bufs = (buf0, buf1)
        idxvs = (idx0v, idx1v)
        wsems = (sw0, sw1)

        jobs = []
        for ty in range(n_t):
            a_hbm, i0_hbm, b_hbm, i1_hbm = ins[4 * ty:4 * ty + 4]
            jobs.append((a_hbm, i0_hbm, outs[2 * ty]))
            jobs.append((b_hbm, i1_hbm, outs[2 * ty + 1]))

        for tab, i_hbm, out_hbm in jobs:
            pend = [None, None]
            ipend = [None, None]
            e00 = pl.multiple_of(base, 8)
            ipend[0] = pltpu.async_copy(i_hbm.at[pl.ds(e00, _SB)], idx0v,
                                        si0)
            for sb in range(_NSB):
                p = sb % 2
                buf = bufs[p]
                if pend[p] is not None:
                    pend[p].wait()
                ipend[p].wait()
                gs = [pltpu.async_copy(
                    tab.at[idxvs[p].at[pl.ds(t * _EB, _EB)]],
                    buf.at[pl.ds(t * _EB, _EB)], sg)
                    for t in range(_SB // _EB)]
                if sb + 1 < _NSB:
                    e1 = pl.multiple_of(base + (sb + 1) * _SB, 8)
                    ipend[1 - p] = pltpu.async_copy(
                        i_hbm.at[pl.ds(e1, _SB)], idxvs[1 - p],
                        isems[1 - p])
                for g in gs:
                    g.wait()
                e0 = pl.multiple_of(base + sb * _SB, 8)
                pend[p] = pltpu.async_copy(
                    buf, out_hbm.at[pl.ds(e0, _SB)], wsems[p])
            for p in range(2):
                if pend[p] is not None:
                    pend[p].wait()
            er = pl.multiple_of(base + _NSB * _SB, 8)
            pltpu.sync_copy(i_hbm.at[pl.ds(er, _REM)],
                            idx0v.at[pl.ds(0, _REM)])
            pltpu.async_copy(tab.at[idx0v.at[pl.ds(0, _REM)]],
                             buf0.at[pl.ds(0, _REM)], sg).wait()
            pltpu.sync_copy(buf0.at[pl.ds(0, _REM)],
                            out_hbm.at[pl.ds(er, _REM)])

    flat = []
    for t4 in tabs_and_idx:
        flat.extend(t4)
    return k(*flat)


_NBPS = _NBLK // _NS
_NBPX = _NBLK - _NBPS * _NS
_CW = _H // _NC
_N0P = 10112
_N0ZR = _N0P // _NS
_N0PAD = 10240
_N1HALF = 10112
_N1ACC = 10240
_N1ZR = _N1ACC // _NS
_N1PAD = 20480


def _scatter_blocks(y_hbm, i_hbm, col0, accum, ibufs, ybufs, lsems, ssems,
                    sub, i_is_2d):
    def load_idx(blk, ib, sem):
        if i_is_2d:
            return pltpu.async_copy(i_hbm.at[blk], ib.at[0], sem)
        return pltpu.async_copy(i_hbm.at[pl.ds(blk * _EB, _EB)], ib.at[0],
                                sem)

    pend = [None, None]
    for j in range(_NBPS):
        p = j % 2
        ib, yb = ibufs[p], ybufs[p]
        if pend[p] is not None:
            pend[p].wait()
        blk = sub * _NBPS + j
        e0 = pl.multiple_of(blk * _EB, _EB)
        la = pltpu.async_copy(y_hbm.at[pl.ds(e0, _EB), pl.ds(col0, _CW)],
                              yb, lsems[p])
        li = load_idx(blk, ib, lsems[p])
        la.wait()
        li.wait()
        pend[p] = pltpu.async_copy(yb, accum.at[ib.at[0]], ssems[p],
                                   add=True)
    for p in range(2):
        if pend[p] is not None:
            pend[p].wait()

    @pl.when(sub < _NBPX)
    def _():
        blk = _NS * _NBPS + sub
        e0 = pl.multiple_of(blk * _EB, _EB)
        pltpu.sync_copy(y_hbm.at[pl.ds(e0, _EB), pl.ds(col0, _CW)],
                        ybufs[0])
        load_idx(blk, ibufs[0], lsems[0]).wait()
        pltpu.sync_copy(ybufs[0], accum.at[ibufs[0].at[0]], add=True)


def _scatter_sweeps_kernel(n_out, out_rows, sweep_spec):
    def make(*arrays):
        @functools.partial(
            pl.kernel,
            out_type=[jax.ShapeDtypeStruct((r, _H), _f32)
                      for r in out_rows],
            mesh=_mesh(),
            scratch_types=[
                pltpu.VMEM((1, _EB), jnp.int32),
                pltpu.VMEM((1, _EB), jnp.int32),
                pltpu.VMEM((_EB, _CW), _f32),
                pltpu.VMEM((_EB, _CW), _f32),
                pltpu.VMEM_SHARED((_N1ACC, _CW), _f32),
                pltpu.SemaphoreType.DMA,
                pltpu.SemaphoreType.DMA,
                pltpu.SemaphoreType.DMA,
                pltpu.SemaphoreType.DMA,
            ],
        )
        def k(*refs):
            nin = len(arrays)
            ins = refs[:nin]
            outs = refs[nin:nin + n_out]
            ib0, ib1, yb0, yb1, accum, sl0, sl1, ss0, ss1 = refs[nin + n_out:]
            core = lax.axis_index("c")
            sub = lax.axis_index("s")
            col0 = pl.multiple_of(core * _CW, _CW)
            r0 = pl.multiple_of(sub * _N1ZR, 8)
            z_hbm = ins[0]

            for yi, ii, is2d, oi, mbase in sweep_spec:
                y_hbm, i_hbm, m_hbm = ins[yi], ins[ii], outs[oi]
                pltpu.sync_copy(z_hbm.at[pl.ds(0, _N1ZR)],
                                accum.at[pl.ds(r0, _N1ZR)])
                plsc.subcore_barrier()
                _scatter_blocks(y_hbm, i_hbm, col0, accum, (ib0, ib1),
                                (yb0, yb1), (sl0, sl1), (ss0, ss1), sub,
                                i_is_2d=is2d)
                plsc.subcore_barrier()

                @pl.when(sub < _NS - 1)
                def _():
                    pltpu.sync_copy(
                        accum.at[pl.ds(r0, _N1ZR)],
                        m_hbm.at[pl.ds(mbase + r0, _N1ZR),
                                 pl.ds(col0, _CW)])

                @pl.when(sub == _NS - 1)
                def _():
                    last = pl.multiple_of((_NS - 1) * _N1ZR, 8)
                    rows = _N1HALF - (_NS - 1) * _N1ZR
                    pltpu.sync_copy(
                        accum.at[pl.ds(last, rows)],
                        m_hbm.at[pl.ds(mbase + last, rows),
                                 pl.ds(col0, _CW)])

                plsc.subcore_barrier()

        return k(*arrays)

    return make


def _sc_scatter_n0_one(y, i, zeros):
    return _scatter_sweeps_kernel(
        1, [_N0PAD], [(1, 2, False, 0, 0)])(zeros, y, i)[0]


def _sc_scatter_n1(y, ilo, ihi, zeros):
    return _scatter_sweeps_kernel(
        1, [_N1PAD], [(1, 2, True, 0, 0), (1, 3, True, 0, _N1HALF)])(
            zeros, y, ilo, ihi)[0]


def _remap_n1(i2d):
    def body(i_ref, lo_ref, hi_ref):
        v = i_ref[...]
        trash = _N1HALF + jnp.bitwise_and(v, 127)
        lo_ref[...] = jnp.where(v < _N1HALF, v, trash)
        hi_ref[...] = jnp.where(v >= _N1HALF, v - _N1HALF, trash)

    return pl.pallas_call(
        body,
        out_shape=[jax.ShapeDtypeStruct((_NBLK, _EB), jnp.int32)] * 2,
    )(i2d)


def _update(parts, w_parts, b1, w2, b2, br=1024):
    n = parts[0].shape[0]
    k = len(parts)

    def body(*refs):
        p_refs = refs[:k]
        wp_refs = refs[k:2 * k]
        b1_ref, w2_ref, b2_ref, o_ref = refs[2 * k:]
        s = jnp.dot(p_refs[0][...], wp_refs[0][...],
                    preferred_element_type=_f32)
        for j in range(1, k):
            s = s + jnp.dot(p_refs[j][...], wp_refs[j][...],
                            preferred_element_type=_f32)
        h = _silu(s + b1_ref[...])
        o_ref[...] = (jnp.dot(h, w2_ref[...], preferred_element_type=_f32)
                      + b2_ref[...] + p_refs[0][...])

    in_specs = [pl.BlockSpec((br, _H), lambda i: (i, 0))] * k
    in_specs += [pl.BlockSpec((_H, _H), lambda i: (0, 0))] * k
    in_specs += [pl.BlockSpec((1, _H), lambda i: (0, 0)),
                 pl.BlockSpec((_H, _H), lambda i: (0, 0)),
                 pl.BlockSpec((1, _H), lambda i: (0, 0))]
    return pl.pallas_call(
        body, grid=(pl.cdiv(n, br),),
        in_specs=in_specs,
        out_specs=pl.BlockSpec((br, _H), lambda i: (i, 0)),
        out_shape=jax.ShapeDtypeStruct((n, _H), _f32),
    )(*parts, *w_parts, b1, w2, b2)


def kernel(x_0, x_1, adj_0_0, adj_1_0, adj_1_1, inv_0_0, inv_1_0, inv_1_1,
           msg_W1_0_0, msg_b1_0_0, msg_W2_0_0, msg_b2_0_0, inf_W_0_0, inf_b_0_0,
           msg_W1_1_0, msg_b1_1_0, msg_W2_1_0, msg_b2_1_0, inf_W_1_0, inf_b_1_0,
           msg_W1_1_1, msg_b1_1_1, msg_W2_1_1, msg_b2_1_1, inf_W_1_1, inf_b_1_1,
           upd_W1_0, upd_b1_0, upd_W2_0, upd_b2_0,
           upd_W1_1, upd_b1_1, upd_W2_1, upd_b2_1):
    i32 = jnp.int32
    idx = {
        "0_0": (adj_0_0[0].astype(i32), adj_0_0[1].astype(i32)),
        "1_0": (adj_1_0[0].astype(i32), adj_1_0[1].astype(i32)),
        "1_1": (adj_1_1[0].astype(i32), adj_1_1[1].astype(i32)),
    }
    w1 = {"0_0": msg_W1_0_0, "1_0": msg_W1_1_0, "1_1": msg_W1_1_1}
    b1 = {"0_0": msg_b1_0_0, "1_0": msg_b1_1_0, "1_1": msg_b1_1_1}
    w2 = {"0_0": msg_W2_0_0, "1_0": msg_W2_1_0, "1_1": msg_W2_1_1}
    b2 = {"0_0": msg_b2_0_0, "1_0": msg_b2_1_0, "1_1": msg_b2_1_1}
    wi = {"0_0": inf_W_0_0, "1_0": inf_W_1_0, "1_1": inf_W_1_1}
    bi = {"0_0": inf_b_0_0, "1_0": inf_b_1_0, "1_1": inf_b_1_1}
    inv = {"0_0": inv_0_0, "1_0": inv_1_0, "1_1": inv_1_1}

    w1s = {a: w1[a][:_H] for a in w1}
    w1r = {a: w1[a][_H:2 * _H] for a in w1}
    w1i = {a: w1[a][2 * _H:] for a in w1}
    b1row = {a: b1[a][None, :] for a in b1}
    b2row = {a: b2[a][None, :] for a in b2}
    w2b = {a: w2[a].astype(jnp.bfloat16) for a in w2}
    wip = {a: jnp.pad(wi[a], ((0, 0), (0, 127))).astype(jnp.bfloat16)
           for a in wi}
    bip = {a: jnp.pad(bi[a], (0, 127))[None, :] for a in bi}
    zrow = jnp.zeros((1, _H), _f32)

    A00, B00, B10 = _proj(
        x_0, [w1s["0_0"], w1r["0_0"], w1r["1_0"]],
        [zrow, b1row["0_0"], b1row["1_0"]])
    A10, A11, B11 = _proj(
        x_1, [w1s["1_0"], w1s["1_1"], w1r["1_1"]],
        [zrow, zrow, b1row["1_1"]])
    tabs = {"0_0": (A00, B00), "1_0": (A10, B10), "1_1": (A11, B11)}

    y = {}
    for a in ("0_0", "1_0", "1_1"):
        ga, gb = _sc_gather_all(
            [(tabs[a][0], idx[a][0], tabs[a][1], idx[a][1])])
        y[a] = _edge_mlp(ga, gb, inv[a], w1i[a], w2b[a],
                         b2row[a], wip[a], bip[a])

    zeros = jnp.zeros((_N1ZR, _CW), _f32)
    ilo, ihi = _remap_n1(idx["1_1"][1].reshape(_NBLK, _EB))
    mes00 = _sc_scatter_n0_one(y["0_0"], idx["0_0"][1], zeros)
    mes10 = _sc_scatter_n0_one(y["1_0"], idx["1_0"][1], zeros)
    mes11 = _sc_scatter_n1(y["1_1"], ilo, ihi, zeros)

    u1_0 = [upd_W1_0[:_H], upd_W1_0[_H:2 * _H], upd_W1_0[2 * _H:]]
    out0 = _update([x_0, mes00, mes10], u1_0, upd_b1_0[None, :],
                   upd_W2_0, upd_b2_0[None, :])
    u1_1 = [upd_W1_1[:_H], upd_W1_1[_H:]]
    out1 = _update([x_1, mes11], u1_1, upd_b1_1[None, :],
                   upd_W2_1, upd_b2_1[None, :])
    return (out0, out1)
